# Initial kernel scaffold; baseline (speedup 1.0000x reference)
import jax
import jax.numpy as jnp
from jax.experimental import pallas as pl


def kernel(op_x, machine_x, edge_precedes, cp_src, cp_dst, cw_src, cw_dst, params):
    raise NotImplementedError("write your pallas kernel here")


# speedup vs baseline: 10.6359x; 10.6359x over previous
import jax
import jax.numpy as jnp
from jax.experimental import pallas as pl

B = 8
N_OP = 50000
N_MCH = 512
OPS_PER = N_OP // B
MCH_PER = N_MCH // B
HID = 32
HEADS = 4
OUT = HID * HEADS
D = OUT // HEADS


def _rows_block(n, pref):
    for b in (pref, 4000, 2000, 1000, 512, 400, 256, 200, 128, 80, 64, 40, 32, 16, 8):
        if b <= n and n % b == 0 and b % 8 == 0:
            return b
    return n


def _full(shape):
    return pl.BlockSpec(shape, lambda i: (0,) * len(shape))


def _embed_ln_kernel(x_ref, w_ref, b_ref, g_ref, bb_ref, o_ref):
    y = jnp.dot(x_ref[...], w_ref[...], preferred_element_type=jnp.float32)
    y = y + b_ref[...]
    mu = jnp.mean(y, axis=-1, keepdims=True)
    var = jnp.mean((y - mu) ** 2, axis=-1, keepdims=True)
    o_ref[...] = (y - mu) / jnp.sqrt(var + 1e-5) * g_ref[...] + bb_ref[...]


def _embed_ln(x, w, b, g, bb):
    n, din = x.shape
    dout = w.shape[1]
    bs = _rows_block(n, 2500)
    return pl.pallas_call(
        _embed_ln_kernel,
        grid=(n // bs,),
        in_specs=[
            pl.BlockSpec((bs, din), lambda i: (i, 0)),
            _full((din, dout)), _full((1, dout)), _full((1, dout)), _full((1, dout)),
        ],
        out_specs=pl.BlockSpec((bs, dout), lambda i: (i, 0)),
        out_shape=jax.ShapeDtypeStruct((n, dout), jnp.float32),
    )(x, w, b.reshape(1, -1), g.reshape(1, -1), bb.reshape(1, -1))


def _matmul_bias_kernel(x_ref, w_ref, b_ref, o_ref):
    o_ref[...] = jnp.dot(x_ref[...], w_ref[...],
                         preferred_element_type=jnp.float32) + b_ref[...]


def _matmul_bias(x, w, b):
    n, din = x.shape
    dout = w.shape[1]
    bs = _rows_block(n, 2500)
    return pl.pallas_call(
        _matmul_bias_kernel,
        grid=(n // bs,),
        in_specs=[
            pl.BlockSpec((bs, din), lambda i: (i, 0)),
            _full((din, dout)), _full((1, dout)),
        ],
        out_specs=pl.BlockSpec((bs, dout), lambda i: (i, 0)),
        out_shape=jax.ShapeDtypeStruct((n, dout), jnp.float32),
    )(x, w, b.reshape(1, -1))


def _kv_rel_kernel(k_ref, v_ref, wk_ref, wv_ref, ko_ref, vo_ref):
    ko_ref[...] = jnp.dot(k_ref[...], wk_ref[...], preferred_element_type=jnp.float32)
    vo_ref[...] = jnp.dot(v_ref[...], wv_ref[...], preferred_element_type=jnp.float32)


def _kv_rel(k, v, wk_bd, wv_bd):
    n = k.shape[0]
    bs = _rows_block(n, 2500)
    return pl.pallas_call(
        _kv_rel_kernel,
        grid=(n // bs,),
        in_specs=[
            pl.BlockSpec((bs, OUT), lambda i: (i, 0)),
            pl.BlockSpec((bs, OUT), lambda i: (i, 0)),
            _full((OUT, OUT)), _full((OUT, OUT)),
        ],
        out_specs=[
            pl.BlockSpec((bs, OUT), lambda i: (i, 0)),
            pl.BlockSpec((bs, OUT), lambda i: (i, 0)),
        ],
        out_shape=[
            jax.ShapeDtypeStruct((n, OUT), jnp.float32),
            jax.ShapeDtypeStruct((n, OUT), jnp.float32),
        ],
    )(k, v, wk_bd, wv_bd)


def _edge_kernel(q_ref, k_ref, v_ref, m_ref, p_ref, ex_ref, wv_ref):
    s = q_ref[...] * k_ref[...]
    a = jnp.dot(s, m_ref[...], preferred_element_type=jnp.float32)
    ex = jnp.exp(a * p_ref[...])
    ex_ref[...] = ex
    wv_ref[...] = v_ref[...] * ex


def _edge_stage(qd, ke, ve, m, pvec):
    e = qd.shape[0]
    bs = _rows_block(e, 4000)
    return pl.pallas_call(
        _edge_kernel,
        grid=(e // bs,),
        in_specs=[
            pl.BlockSpec((bs, OUT), lambda i: (i, 0)),
            pl.BlockSpec((bs, OUT), lambda i: (i, 0)),
            pl.BlockSpec((bs, OUT), lambda i: (i, 0)),
            _full((OUT, OUT)), _full((1, OUT)),
        ],
        out_specs=[
            pl.BlockSpec((bs, OUT), lambda i: (i, 0)),
            pl.BlockSpec((bs, OUT), lambda i: (i, 0)),
        ],
        out_shape=[
            jax.ShapeDtypeStruct((e, OUT), jnp.float32),
            jax.ShapeDtypeStruct((e, OUT), jnp.float32),
        ],
    )(qd, ke, ve, m, pvec)


def _out_kernel(num_ref, den_ref, x_ref, w_ref, b_ref, s_ref, g_ref, bb_ref, o_ref):
    agg = num_ref[...] / (den_ref[...] + 1e-16)
    o = jnp.dot(jax.nn.gelu(agg), w_ref[...],
                preferred_element_type=jnp.float32) + b_ref[...]
    s = s_ref[0, 0]
    y = s * o + (1.0 - s) * x_ref[...] + x_ref[...]
    mu = jnp.mean(y, axis=-1, keepdims=True)
    var = jnp.mean((y - mu) ** 2, axis=-1, keepdims=True)
    o_ref[...] = (y - mu) / jnp.sqrt(var + 1e-5) * g_ref[...] + bb_ref[...]


def _out_stage(num, den, x, w, b, s, g, bb):
    n = x.shape[0]
    bs = _rows_block(n, 2500)
    return pl.pallas_call(
        _out_kernel,
        grid=(n // bs,),
        in_specs=[
            pl.BlockSpec((bs, OUT), lambda i: (i, 0)),
            pl.BlockSpec((bs, OUT), lambda i: (i, 0)),
            pl.BlockSpec((bs, OUT), lambda i: (i, 0)),
            _full((OUT, OUT)), _full((1, OUT)), _full((1, 1)),
            _full((1, OUT)), _full((1, OUT)),
        ],
        out_specs=pl.BlockSpec((bs, OUT), lambda i: (i, 0)),
        out_shape=jax.ShapeDtypeStruct((n, OUT), jnp.float32),
    )(num, den, x, w, b.reshape(1, -1), s.reshape(1, 1),
      g.reshape(1, -1), bb.reshape(1, -1))


def _pool_kernel(m_ref, x_ref, o_ref):
    @pl.when(pl.program_id(0) == 0)
    def _():
        o_ref[...] = jnp.zeros_like(o_ref)
    o_ref[...] += jax.lax.dot_general(
        m_ref[...], x_ref[...], (((0,), (0,)), ((), ())),
        preferred_element_type=jnp.float32)


def _pool(x, nb):
    n = x.shape[0]
    per = n // nb
    pm = (jnp.repeat(jnp.arange(nb), per)[:, None] ==
          jnp.arange(nb)[None, :]).astype(jnp.float32) / float(per)
    bs = _rows_block(n, 2000)
    return pl.pallas_call(
        _pool_kernel,
        grid=(n // bs,),
        in_specs=[
            pl.BlockSpec((bs, nb), lambda i: (i, 0)),
            pl.BlockSpec((bs, OUT), lambda i: (i, 0)),
        ],
        out_specs=pl.BlockSpec((nb, OUT), lambda i: (0, 0)),
        out_shape=jax.ShapeDtypeStruct((nb, OUT), jnp.float32),
    )(pm, x)


def _block_diag(mats):
    out = jnp.zeros((OUT, OUT), jnp.float32)
    for h in range(HEADS):
        out = out.at[h * D:(h + 1) * D, h * D:(h + 1) * D].set(mats[h])
    return out


def kernel(op_x, machine_x, edge_precedes, cp_src, cp_dst, cw_src, cw_dst, params):
    hb = jnp.arange(OUT) // D
    m_head = (hb[:, None] == hb[None, :]).astype(jnp.float32)

    op_h = _embed_ln(op_x, params["emb_op_w"], params["emb_op_b"],
                     params["op_norm_g"], params["op_norm_b"])
    mch_h = _embed_ln(machine_x, params["emb_mch_w"], params["emb_mch_b"],
                      params["mch_norm_g"], params["mch_norm_b"])

    src1, dst1 = edge_precedes[0], edge_precedes[1]
    dst_op = jnp.concatenate([dst1, cp_dst])

    x = {"op": op_h, "machine": mch_h}
    for lp in params["layers"]:
        kqv = {}
        for t in ("op", "machine"):
            full = _matmul_bias(x[t], lp["w_kqv"][t], lp["b_kqv"][t])
            kqv[t] = (full[:, :OUT], full[:, OUT:2 * OUT], full[:, 2 * OUT:])
        k_op, q_op, v_op = kqv["op"]
        k_m, q_m, v_m = kqv["machine"]

        sc = 1.0 / jnp.sqrt(float(D))
        pv = {r: jnp.repeat(lp["p_rel"][r] * sc, D).reshape(1, OUT)
              for r in ("precedes", "can_process", "compatible_with")}

        kr1, vr1 = _kv_rel(k_op, v_op,
                           _block_diag(lp["k_rel"]["precedes"]),
                           _block_diag(lp["v_rel"]["precedes"]))
        kr2, vr2 = _kv_rel(k_m, v_m,
                           _block_diag(lp["k_rel"]["can_process"]),
                           _block_diag(lp["v_rel"]["can_process"]))
        kr3, vr3 = _kv_rel(k_op, v_op,
                           _block_diag(lp["k_rel"]["compatible_with"]),
                           _block_diag(lp["v_rel"]["compatible_with"]))

        ex1, wv1 = _edge_stage(q_op[dst1], kr1[src1], vr1[src1], m_head, pv["precedes"])
        ex2, wv2 = _edge_stage(q_op[cp_dst], kr2[cp_src], vr2[cp_src], m_head, pv["can_process"])
        ex3, wv3 = _edge_stage(q_m[cw_dst], kr3[cw_src], vr3[cw_src], m_head, pv["compatible_with"])

        ex_op = jnp.concatenate([ex1, ex2])
        wv_op = jnp.concatenate([wv1, wv2])
        den_op = jax.ops.segment_sum(ex_op, dst_op, num_segments=N_OP)
        num_op = jax.ops.segment_sum(wv_op, dst_op, num_segments=N_OP)
        den_m = jax.ops.segment_sum(ex3, cw_dst, num_segments=N_MCH)
        num_m = jax.ops.segment_sum(wv3, cw_dst, num_segments=N_MCH)

        new_x = {}
        s_op = jax.nn.sigmoid(lp["skip"]["op"])
        s_m = jax.nn.sigmoid(lp["skip"]["machine"])
        new_x["op"] = _out_stage(num_op, den_op, x["op"], lp["w_out"]["op"],
                                 lp["b_out"]["op"], s_op, lp["ln_g"], lp["ln_b"])
        new_x["machine"] = _out_stage(num_m, den_m, x["machine"], lp["w_out"]["machine"],
                                      lp["b_out"]["machine"], s_m, lp["ln_g"], lp["ln_b"])
        x = new_x

    fea_j_global = _pool(x["op"], B)
    fea_m_global = _pool(x["machine"], B)
    fea_j = x["op"].reshape(B, OPS_PER, OUT)
    fea_m = x["machine"].reshape(B, MCH_PER, OUT)
    return (fea_j, fea_m, fea_j_global[:, None, :], fea_m_global[:, None, :])
